# baseline (speedup 1.0000x reference)
import functools

import jax
import jax.numpy as jnp
from jax import lax
from jax.experimental import pallas as pl
from jax.experimental.pallas import tpu as pltpu
from jax.experimental.pallas import tpu_sc as plsc

SEQ = 2048
HID = 1024
N_BANDS = 5
K_KEEP = SEQ * HID // 2
ROW_CHUNK = 128
N_CHUNKS = SEQ // ROW_CHUNK
BAND_ROWS = ((0, 128), (128, 128), (256, 256), (512, 512), (1024, 1024))

N_BINS = 4096
BIN_SHIFT = 19
LANES = 16
SC_SUBCORES = 16
SC_ROWS = SEQ // SC_SUBCORES
SC_BUF_ROWS = 16
SC_N_CHUNKS = SC_ROWS // SC_BUF_ROWS


def _band_of_chunk(t):
    bid = jnp.where(t >= 1, 1, 0)
    bid = jnp.where(t >= 2, 2, bid)
    bid = jnp.where(t >= 4, 3, bid)
    bid = jnp.where(t >= 8, 4, bid)
    return bid


def _band_row(bimp_sig, bid):
    row = bimp_sig[0]
    for i in range(1, N_BANDS):
        row = jnp.where(bid == i, bimp_sig[i], row)
    return row


def _sc_hist_kernel(coeffs_hbm, bimp_hbm, dimp_hbm, hist_hbm,
                    bimp_v, dimp_v, row_v, buf_v, hist_v, hsum_v):
    c = lax.axis_index("c")
    s = lax.axis_index("s")
    bid = _band_of_chunk(s)

    pltpu.sync_copy(bimp_hbm, bimp_v)
    pltpu.sync_copy(dimp_hbm, dimp_v)

    def row_body(j, _):
        sl = pl.ds(j * LANES, LANES)
        bsel = bimp_v[0, sl]
        for i in range(1, N_BANDS):
            bsel = jnp.where(bid == i, bimp_v[i, sl], bsel)
        bs = 1.0 / (1.0 + jnp.exp(-bsel))
        ds_ = 1.0 / (1.0 + jnp.exp(-dimp_v[sl]))
        row_v[sl] = bs * ds_
        return 0
    lax.fori_loop(0, HID // LANES, row_body, 0)

    zeros = jnp.zeros((LANES,), jnp.int32)

    def zero_body(i, _):
        for u in range(8):
            hist_v[pl.ds((i * 8 + u) * LANES, LANES)] = zeros
        return 0
    lax.fori_loop(0, N_BINS * LANES // LANES // 8, zero_body, 0)

    shiftv = jnp.full((LANES,), BIN_SHIFT, jnp.int32)
    ones = jnp.ones((LANES,), jnp.int32)
    lane_off = lax.broadcasted_iota(jnp.int32, (LANES,), 0) * N_BINS

    def chunk_body(ch, _):
        pltpu.sync_copy(
            coeffs_hbm.at[c, pl.ds(s * SC_ROWS + ch * SC_BUF_ROWS,
                                   SC_BUF_ROWS), :],
            buf_v)

        def row_body(r, _):
            for j in range(HID // LANES):
                sl = pl.ds(j * LANES, LANES)
                ci = jnp.abs(buf_v[r, sl]) * row_v[sl]
                bits = lax.bitcast_convert_type(ci, jnp.int32)
                idx = lax.shift_right_logical(bits, shiftv) + lane_off
                plsc.addupdate_scatter(hist_v, [idx], ones)
            return 0
        lax.fori_loop(0, SC_BUF_ROWS, row_body, 0)
        return 0
    lax.fori_loop(0, SC_N_CHUNKS, chunk_body, 0)

    def merge_body(i, _):
        sl = pl.ds(i * LANES, LANES)
        acc = hist_v[sl]
        for l in range(1, LANES):
            acc = acc + hist_v[pl.ds(l * N_BINS + i * LANES, LANES)]
        hsum_v[sl] = acc
        return 0
    lax.fori_loop(0, N_BINS // LANES, merge_body, 0)

    pltpu.sync_copy(hsum_v, hist_hbm.at[c, s])


def _select_kernel(hist_ref, thr_ref):
    merged = jnp.sum(hist_ref[0], axis=0)
    rows = lax.broadcasted_iota(jnp.int32, (N_BINS // 128, 128), 0)
    cols = lax.broadcasted_iota(jnp.int32, (N_BINS // 128, 128), 1)
    bin_ids = rows * 128 + cols

    def cnt_ge(b):
        return jnp.sum(jnp.where(bin_ids >= b, merged, 0))

    def bisect(_, lohi):
        lo, hi = lohi
        mid = (lo + hi) // 2
        ge = cnt_ge(mid) >= K_KEEP
        return jnp.where(ge, mid, lo), jnp.where(ge, hi, mid)

    lo, hi = lax.fori_loop(0, 12, bisect,
                           (jnp.zeros((), jnp.int32),
                            jnp.full((), N_BINS, jnp.int32)))
    bits = (jnp.full((1, 1), 1, jnp.int32) << (BIN_SHIFT - 1)) \
        + (lo << BIN_SHIFT)
    thr = lax.bitcast_convert_type(bits, jnp.float32)
    thr_ref[...] = thr.reshape(1, 1, 1)


def _mask_kernel(thr_ref, coeffs_ref, bimp_ref, dimp_ref, temp_ref,
                 filt_ref, mask_ref, imp_ref):
    t = pl.program_id(1)
    dimp = jax.nn.sigmoid(dimp_ref[0, :])
    bimp_sig = jax.nn.sigmoid(bimp_ref[...]) * dimp[None, :]
    row = _band_row(bimp_sig, _band_of_chunk(t))

    thr = thr_ref[0, 0, 0]
    inv_temp = 1.0 / jnp.abs(temp_ref[0, 0])
    c = coeffs_ref[0]
    ci = jnp.abs(c) * row[None, :]
    m = jax.nn.sigmoid((ci - thr) * inv_temp)
    filt_ref[0] = c * m
    mask_ref[0] = m
    imp_ref[0] = jnp.broadcast_to(row[None, :], (ROW_CHUNK, HID))


def kernel(coeffs, band_importance, dim_importance, temperature):
    B = coeffs.shape[0]
    dimp1 = dim_importance.astype(jnp.float32)
    dimp2 = dim_importance.reshape(1, HID).astype(jnp.float32)
    temp2 = jnp.reshape(temperature, (1, 1)).astype(jnp.float32)

    sc_hist = functools.partial(
        pl.kernel,
        mesh=plsc.VectorSubcoreMesh(core_axis_name="c", subcore_axis_name="s",
                                    num_cores=2, num_subcores=SC_SUBCORES),
        out_type=jax.ShapeDtypeStruct((B, SC_SUBCORES, N_BINS), jnp.int32),
        scratch_types=[
            pltpu.VMEM((N_BANDS, HID), jnp.float32),
            pltpu.VMEM((HID,), jnp.float32),
            pltpu.VMEM((HID,), jnp.float32),
            pltpu.VMEM((SC_BUF_ROWS, HID), jnp.float32),
            pltpu.VMEM((N_BINS * LANES,), jnp.int32),
            pltpu.VMEM((N_BINS,), jnp.int32),
        ],
        compiler_params=pltpu.CompilerParams(needs_layout_passes=False),
    )(_sc_hist_kernel)
    hist = sc_hist(coeffs, band_importance, dimp1)

    thr = pl.pallas_call(
        _select_kernel,
        grid=(B,),
        in_specs=[
            pl.BlockSpec((1, SC_SUBCORES, N_BINS // 128, 128),
                         lambda b: (b, 0, 0, 0)),
        ],
        out_specs=pl.BlockSpec((1, 1, 1), lambda b: (b, 0, 0)),
        out_shape=jax.ShapeDtypeStruct((B, 1, 1), jnp.float32),
    )(hist.reshape(B, SC_SUBCORES, N_BINS // 128, 128))

    filt, mask, imp = pl.pallas_call(
        _mask_kernel,
        grid=(B, N_CHUNKS),
        in_specs=[
            pl.BlockSpec((1, 1, 1), lambda b, t: (b, 0, 0)),
            pl.BlockSpec((1, ROW_CHUNK, HID), lambda b, t: (b, t, 0)),
            pl.BlockSpec((N_BANDS, HID), lambda b, t: (0, 0)),
            pl.BlockSpec((1, HID), lambda b, t: (0, 0)),
            pl.BlockSpec((1, 1), lambda b, t: (0, 0)),
        ],
        out_specs=[
            pl.BlockSpec((1, ROW_CHUNK, HID), lambda b, t: (b, t, 0)),
            pl.BlockSpec((1, ROW_CHUNK, HID), lambda b, t: (b, t, 0)),
            pl.BlockSpec((1, ROW_CHUNK, HID), lambda b, t: (b, t, 0)),
        ],
        out_shape=[
            jax.ShapeDtypeStruct((B, SEQ, HID), jnp.float32),
            jax.ShapeDtypeStruct((B, SEQ, HID), jnp.float32),
            jax.ShapeDtypeStruct((B, SEQ, HID), jnp.float32),
        ],
        compiler_params=pltpu.CompilerParams(
            dimension_semantics=("parallel", "arbitrary")),
    )(thr, coeffs, band_importance, dimp2, temp2)

    return (filt, mask, imp)


# speedup vs baseline: 1.0957x; 1.0957x over previous
import functools

import jax
import jax.numpy as jnp
from jax import lax
from jax.experimental import pallas as pl
from jax.experimental.pallas import tpu as pltpu
from jax.experimental.pallas import tpu_sc as plsc

SEQ = 2048
HID = 1024
N_BANDS = 5
K_KEEP = SEQ * HID // 2
ROW_CHUNK = 128
N_CHUNKS = SEQ // ROW_CHUNK
BAND_ROWS = ((0, 128), (128, 128), (256, 256), (512, 512), (1024, 1024))

N_BINS = 32768
BIN_SHIFT = 16
LANES = 16
SC_SUBCORES = 16
SC_ROWS = SEQ // SC_SUBCORES
SC_BUF_ROWS = 16
SC_N_CHUNKS = SC_ROWS // SC_BUF_ROWS


def _band_of_chunk(t):
    bid = jnp.where(t >= 1, 1, 0)
    bid = jnp.where(t >= 2, 2, bid)
    bid = jnp.where(t >= 4, 3, bid)
    bid = jnp.where(t >= 8, 4, bid)
    return bid


def _band_row(bimp_sig, bid):
    row = bimp_sig[0]
    for i in range(1, N_BANDS):
        row = jnp.where(bid == i, bimp_sig[i], row)
    return row


def _sc_hist_kernel(coeffs_hbm, bimp_hbm, dimp_hbm, hist_hbm,
                    bimp_v, dimp_v, row_v, buf_v, hist_v):
    c = lax.axis_index("c")
    s = lax.axis_index("s")
    bid = _band_of_chunk(s)

    pltpu.sync_copy(bimp_hbm, bimp_v)
    pltpu.sync_copy(dimp_hbm, dimp_v)

    def row_body(j, _):
        sl = pl.ds(j * LANES, LANES)
        bsel = bimp_v[0, sl]
        for i in range(1, N_BANDS):
            bsel = jnp.where(bid == i, bimp_v[i, sl], bsel)
        bs = 1.0 / (1.0 + jnp.exp(-bsel))
        ds_ = 1.0 / (1.0 + jnp.exp(-dimp_v[sl]))
        row_v[sl] = bs * ds_
        return 0
    lax.fori_loop(0, HID // LANES, row_body, 0)

    zeros = jnp.zeros((LANES,), jnp.int32)

    def zero_body(i, _):
        for u in range(8):
            hist_v[pl.ds((i * 8 + u) * LANES, LANES)] = zeros
        return 0
    lax.fori_loop(0, N_BINS // LANES // 8, zero_body, 0)

    shiftv = jnp.full((LANES,), BIN_SHIFT, jnp.int32)
    ones = jnp.ones((LANES,), jnp.int32)

    def chunk_body(ch, _):
        pltpu.sync_copy(
            coeffs_hbm.at[c, pl.ds(s * SC_ROWS + ch * SC_BUF_ROWS,
                                   SC_BUF_ROWS), :],
            buf_v)

        def col_body(j, _):
            sl = pl.ds(j * LANES, LANES)
            row16 = row_v[sl]
            for r in range(SC_BUF_ROWS):
                ci = jnp.abs(buf_v[r, sl]) * row16
                bits = lax.bitcast_convert_type(ci, jnp.int32)
                idx = lax.shift_right_logical(bits, shiftv)
                plsc.addupdate_scatter(hist_v, [idx], ones)
            return 0
        lax.fori_loop(0, HID // LANES, col_body, 0)
        return 0
    lax.fori_loop(0, SC_N_CHUNKS, chunk_body, 0)

    pltpu.sync_copy(hist_v, hist_hbm.at[c, s])


def _select_kernel(hist_ref, thr_ref):
    merged = jnp.sum(hist_ref[0], axis=0)
    rows = lax.broadcasted_iota(jnp.int32, (N_BINS // 128, 128), 0)
    cols = lax.broadcasted_iota(jnp.int32, (N_BINS // 128, 128), 1)
    bin_ids = rows * 128 + cols

    def cnt_ge(b):
        return jnp.sum(jnp.where(bin_ids >= b, merged, 0))

    def bisect(_, lohi):
        lo, hi = lohi
        mid = (lo + hi) // 2
        ge = cnt_ge(mid) >= K_KEEP
        return jnp.where(ge, mid, lo), jnp.where(ge, hi, mid)

    lo, hi = lax.fori_loop(0, 15, bisect,
                           (jnp.zeros((), jnp.int32),
                            jnp.full((), N_BINS, jnp.int32)))
    bits = (jnp.full((1, 1), 1, jnp.int32) << (BIN_SHIFT - 1)) \
        + (lo << BIN_SHIFT)
    thr = lax.bitcast_convert_type(bits, jnp.float32)
    thr_ref[...] = thr.reshape(1, 1, 1)


def _mask_kernel(thr_ref, coeffs_ref, bimp_ref, dimp_ref, temp_ref,
                 filt_ref, mask_ref, imp_ref):
    t = pl.program_id(1)
    dimp = jax.nn.sigmoid(dimp_ref[0, :])
    bimp_sig = jax.nn.sigmoid(bimp_ref[...]) * dimp[None, :]
    row = _band_row(bimp_sig, _band_of_chunk(t))

    thr = thr_ref[0, 0, 0]
    inv_temp = 1.0 / jnp.abs(temp_ref[0, 0])
    c = coeffs_ref[0]
    ci = jnp.abs(c) * row[None, :]
    m = jax.nn.sigmoid((ci - thr) * inv_temp)
    filt_ref[0] = c * m
    mask_ref[0] = m
    imp_ref[0] = jnp.broadcast_to(row[None, :], (ROW_CHUNK, HID))


def kernel(coeffs, band_importance, dim_importance, temperature):
    B = coeffs.shape[0]
    dimp1 = dim_importance.astype(jnp.float32)
    dimp2 = dim_importance.reshape(1, HID).astype(jnp.float32)
    temp2 = jnp.reshape(temperature, (1, 1)).astype(jnp.float32)

    sc_hist = functools.partial(
        pl.kernel,
        mesh=plsc.VectorSubcoreMesh(core_axis_name="c", subcore_axis_name="s",
                                    num_cores=2, num_subcores=SC_SUBCORES),
        out_type=jax.ShapeDtypeStruct((B, SC_SUBCORES, N_BINS), jnp.int32),
        scratch_types=[
            pltpu.VMEM((N_BANDS, HID), jnp.float32),
            pltpu.VMEM((HID,), jnp.float32),
            pltpu.VMEM((HID,), jnp.float32),
            pltpu.VMEM((SC_BUF_ROWS, HID), jnp.float32),
            pltpu.VMEM((N_BINS,), jnp.int32),
        ],
        compiler_params=pltpu.CompilerParams(needs_layout_passes=False),
    )(_sc_hist_kernel)
    hist = sc_hist(coeffs, band_importance, dimp1)

    thr = pl.pallas_call(
        _select_kernel,
        grid=(B,),
        in_specs=[
            pl.BlockSpec((1, SC_SUBCORES, N_BINS // 128, 128),
                         lambda b: (b, 0, 0, 0)),
        ],
        out_specs=pl.BlockSpec((1, 1, 1), lambda b: (b, 0, 0)),
        out_shape=jax.ShapeDtypeStruct((B, 1, 1), jnp.float32),
    )(hist.reshape(B, SC_SUBCORES, N_BINS // 128, 128))

    filt, mask, imp = pl.pallas_call(
        _mask_kernel,
        grid=(B, N_CHUNKS),
        in_specs=[
            pl.BlockSpec((1, 1, 1), lambda b, t: (b, 0, 0)),
            pl.BlockSpec((1, ROW_CHUNK, HID), lambda b, t: (b, t, 0)),
            pl.BlockSpec((N_BANDS, HID), lambda b, t: (0, 0)),
            pl.BlockSpec((1, HID), lambda b, t: (0, 0)),
            pl.BlockSpec((1, 1), lambda b, t: (0, 0)),
        ],
        out_specs=[
            pl.BlockSpec((1, ROW_CHUNK, HID), lambda b, t: (b, t, 0)),
            pl.BlockSpec((1, ROW_CHUNK, HID), lambda b, t: (b, t, 0)),
            pl.BlockSpec((1, ROW_CHUNK, HID), lambda b, t: (b, t, 0)),
        ],
        out_shape=[
            jax.ShapeDtypeStruct((B, SEQ, HID), jnp.float32),
            jax.ShapeDtypeStruct((B, SEQ, HID), jnp.float32),
            jax.ShapeDtypeStruct((B, SEQ, HID), jnp.float32),
        ],
        compiler_params=pltpu.CompilerParams(
            dimension_semantics=("parallel", "arbitrary")),
    )(thr, coeffs, band_importance, dimp2, temp2)

    return (filt, mask, imp)


# speedup vs baseline: 1.0967x; 1.0010x over previous
import functools

import jax
import jax.numpy as jnp
from jax import lax
from jax.experimental import pallas as pl
from jax.experimental.pallas import tpu as pltpu
from jax.experimental.pallas import tpu_sc as plsc

SEQ = 2048
HID = 1024
N_BANDS = 5
K_KEEP = SEQ * HID // 2
ROW_CHUNK = 128
N_CHUNKS = SEQ // ROW_CHUNK

N_BINS = 32768
BIN_SHIFT = 16
LANES = 16
SC_SUBCORES = 16
SC_ROWS = SEQ // SC_SUBCORES
SC_BUF_ROWS = 16
SC_N_CHUNKS = SC_ROWS // SC_BUF_ROWS


def _band_of_chunk(t):
    bid = jnp.where(t >= 1, 1, 0)
    bid = jnp.where(t >= 2, 2, bid)
    bid = jnp.where(t >= 4, 3, bid)
    bid = jnp.where(t >= 8, 4, bid)
    return bid


def _band_row(bimp_sig, bid):
    row = bimp_sig[0]
    for i in range(1, N_BANDS):
        row = jnp.where(bid == i, bimp_sig[i], row)
    return row


def _sc_hist_kernel(coeffs_hbm, bimp_hbm, dimp_hbm, hist_hbm,
                    bimp_v, dimp_v, row_v, buf_v, hist_v):
    c = lax.axis_index("c")
    s = lax.axis_index("s")
    bid = _band_of_chunk(s)

    pltpu.sync_copy(bimp_hbm, bimp_v)
    pltpu.sync_copy(dimp_hbm, dimp_v)

    def row_body(j, _):
        sl = pl.ds(j * LANES, LANES)
        bsel = bimp_v[0, sl]
        for i in range(1, N_BANDS):
            bsel = jnp.where(bid == i, bimp_v[i, sl], bsel)
        bs = 1.0 / (1.0 + jnp.exp(-bsel))
        ds_ = 1.0 / (1.0 + jnp.exp(-dimp_v[sl]))
        row_v[sl] = bs * ds_
        return 0
    lax.fori_loop(0, HID // LANES, row_body, 0)

    zeros = jnp.zeros((LANES,), jnp.int32)

    def zero_body(i, _):
        for u in range(8):
            hist_v[pl.ds((i * 8 + u) * LANES, LANES)] = zeros
        return 0
    lax.fori_loop(0, N_BINS // LANES // 8, zero_body, 0)

    shiftv = jnp.full((LANES,), BIN_SHIFT, jnp.int32)
    ones = jnp.ones((LANES,), jnp.int32)

    def chunk_body(ch, _):
        pltpu.sync_copy(
            coeffs_hbm.at[c, pl.ds(s * SC_ROWS + ch * SC_BUF_ROWS,
                                   SC_BUF_ROWS), :],
            buf_v)

        def col_body(j, _):
            sl = pl.ds(j * LANES, LANES)
            row16 = row_v[sl]
            for r in range(SC_BUF_ROWS):
                ci = jnp.abs(buf_v[r, sl]) * row16
                bits = lax.bitcast_convert_type(ci, jnp.int32)
                idx = lax.shift_right_logical(bits, shiftv)
                plsc.addupdate_scatter(hist_v, [idx], ones)
            return 0
        lax.fori_loop(0, HID // LANES, col_body, 0)
        return 0
    lax.fori_loop(0, SC_N_CHUNKS, chunk_body, 0)

    pltpu.sync_copy(hist_v, hist_hbm.at[c, s])


def _select_kernel(hist_ref, thr_ref):
    merged = jnp.sum(hist_ref[0], axis=0)
    rows = lax.broadcasted_iota(jnp.int32, (N_BINS // 128, 128), 0)
    cols = lax.broadcasted_iota(jnp.int32, (N_BINS // 128, 128), 1)
    bin_ids = rows * 128 + cols

    def cnt_ge(b):
        return jnp.sum(jnp.where(bin_ids >= b, merged, 0))

    def bisect(_, lohi):
        lo, hi = lohi
        mid = (lo + hi) // 2
        ge = cnt_ge(mid) >= K_KEEP
        return jnp.where(ge, mid, lo), jnp.where(ge, hi, mid)

    lo, hi = lax.fori_loop(0, 15, bisect,
                           (jnp.zeros((), jnp.int32),
                            jnp.full((), N_BINS, jnp.int32)))
    bits = (jnp.full((1, 1), 1, jnp.int32) << (BIN_SHIFT - 1)) \
        + (lo << BIN_SHIFT)
    thr = lax.bitcast_convert_type(bits, jnp.float32)
    thr_ref[...] = thr.reshape(1, 1, 1)


def _mask_kernel(thr_ref, coeffs_ref, bimp_ref, dimp_ref, temp_ref,
                 filt_ref, mask_ref, imp_ref):
    t = pl.program_id(1)
    dimp = jax.nn.sigmoid(dimp_ref[0, :])
    bimp_sig = jax.nn.sigmoid(bimp_ref[...]) * dimp[None, :]
    row = _band_row(bimp_sig, _band_of_chunk(t))

    thr = thr_ref[0, 0, 0]
    inv_temp = 1.0 / jnp.abs(temp_ref[0, 0])
    c = coeffs_ref[0]
    ci = jnp.abs(c) * row[None, :]
    m = jax.nn.sigmoid((ci - thr) * inv_temp)
    filt_ref[0] = c * m
    mask_ref[0] = m
    imp_ref[0] = jnp.broadcast_to(row[None, :], (ROW_CHUNK, HID))


def kernel(coeffs, band_importance, dim_importance, temperature):
    B = coeffs.shape[0]
    dimp1 = dim_importance.astype(jnp.float32)
    dimp2 = dim_importance.reshape(1, HID).astype(jnp.float32)
    temp2 = jnp.reshape(temperature, (1, 1)).astype(jnp.float32)

    sc_hist = functools.partial(
        pl.kernel,
        mesh=plsc.VectorSubcoreMesh(core_axis_name="c", subcore_axis_name="s",
                                    num_cores=2, num_subcores=SC_SUBCORES),
        out_type=jax.ShapeDtypeStruct((B, SC_SUBCORES, N_BINS), jnp.int32),
        scratch_types=[
            pltpu.VMEM((N_BANDS, HID), jnp.float32),
            pltpu.VMEM((HID,), jnp.float32),
            pltpu.VMEM((HID,), jnp.float32),
            pltpu.VMEM((SC_BUF_ROWS, HID), jnp.float32),
            pltpu.VMEM((N_BINS,), jnp.int32),
        ],
        compiler_params=pltpu.CompilerParams(needs_layout_passes=False),
    )(_sc_hist_kernel)
    hist = sc_hist(coeffs, band_importance, dimp1)

    thr = pl.pallas_call(
        _select_kernel,
        grid=(B,),
        in_specs=[
            pl.BlockSpec((1, SC_SUBCORES, N_BINS // 128, 128),
                         lambda b: (b, 0, 0, 0)),
        ],
        out_specs=pl.BlockSpec((1, 1, 1), lambda b: (b, 0, 0)),
        out_shape=jax.ShapeDtypeStruct((B, 1, 1), jnp.float32),
    )(hist.reshape(B, SC_SUBCORES, N_BINS // 128, 128))

    filt, mask, imp = pl.pallas_call(
        _mask_kernel,
        grid=(B, N_CHUNKS),
        in_specs=[
            pl.BlockSpec((1, 1, 1), lambda b, t: (b, 0, 0)),
            pl.BlockSpec((1, ROW_CHUNK, HID), lambda b, t: (b, t, 0)),
            pl.BlockSpec((N_BANDS, HID), lambda b, t: (0, 0)),
            pl.BlockSpec((1, HID), lambda b, t: (0, 0)),
            pl.BlockSpec((1, 1), lambda b, t: (0, 0)),
        ],
        out_specs=[
            pl.BlockSpec((1, ROW_CHUNK, HID), lambda b, t: (b, t, 0)),
            pl.BlockSpec((1, ROW_CHUNK, HID), lambda b, t: (b, t, 0)),
            pl.BlockSpec((1, ROW_CHUNK, HID), lambda b, t: (b, t, 0)),
        ],
        out_shape=[
            jax.ShapeDtypeStruct((B, SEQ, HID), jnp.float32),
            jax.ShapeDtypeStruct((B, SEQ, HID), jnp.float32),
            jax.ShapeDtypeStruct((B, SEQ, HID), jnp.float32),
        ],
        compiler_params=pltpu.CompilerParams(
            dimension_semantics=("parallel", "arbitrary")),
    )(thr, coeffs, band_importance, dimp2, temp2)

    return (filt, mask, imp)
